# baseline (speedup 1.0000x reference)
import functools

import jax
import jax.numpy as jnp
from jax import lax
from jax.experimental import pallas as pl
from jax.experimental.pallas import tpu as pltpu
from jax.experimental.pallas import tpu_sc as plsc

_V = 1000
_D = 1000
_B, _T = 1024, 50
_N = _B * _T
_NC, _NS = 2, 16
_NW = _NC * _NS
_PER_W = _N // _NW
_CHUNK = 64
_NCHUNK = _PER_W // _CHUNK


def _gather_body(table_hbm, idx_hbm, out_hbm, idx_v, rows_v, gsems, wsems):
    wid = lax.axis_index("s") * _NC + lax.axis_index("c")
    base = wid * _PER_W
    pltpu.sync_copy(idx_hbm.at[pl.ds(base, _PER_W)], idx_v)

    def fire_gather(ci):
        pltpu.async_copy(
            table_hbm.at[idx_v.at[pl.ds(ci * _CHUNK, _CHUNK)]],
            rows_v.at[ci % 2],
            gsems.at[ci % 2],
        )

    def wait_gather(ci):
        pltpu.make_async_copy(
            table_hbm.at[idx_v.at[pl.ds(ci * _CHUNK, _CHUNK)]],
            rows_v.at[ci % 2],
            gsems.at[ci % 2],
        ).wait()

    def fire_write(ci):
        pltpu.async_copy(
            rows_v.at[ci % 2],
            out_hbm.at[pl.ds(base + ci * _CHUNK, _CHUNK)],
            wsems.at[ci % 2],
        )

    def wait_write(ci):
        pltpu.make_async_copy(
            rows_v.at[ci % 2],
            out_hbm.at[pl.ds(base + ci * _CHUNK, _CHUNK)],
            wsems.at[ci % 2],
        ).wait()

    fire_gather(0)
    fire_gather(1)
    for ci in range(_NCHUNK):
        wait_gather(ci)
        fire_write(ci)
        if ci + 2 < _NCHUNK:
            wait_write(ci)
            fire_gather(ci + 2)
    wait_write(_NCHUNK - 2)
    wait_write(_NCHUNK - 1)


_mesh = plsc.VectorSubcoreMesh(core_axis_name="c", subcore_axis_name="s")

_gather = functools.partial(
    pl.kernel,
    out_type=jax.ShapeDtypeStruct((_N, _D), jnp.float32),
    mesh=_mesh,
    scratch_types=[
        pltpu.VMEM((_PER_W,), jnp.int32),
        pltpu.VMEM((2, _CHUNK, _D), jnp.float32),
        pltpu.SemaphoreType.DMA((2,)),
        pltpu.SemaphoreType.DMA((2,)),
    ],
    compiler_params=pltpu.CompilerParams(use_tc_tiling_on_sc=False),
)(_gather_body)


@jax.jit
def kernel(x, embeddings):
    idx = x.reshape(_N)
    out = _gather(embeddings, idx)
    return out.reshape(_B, _T, _D)


# speedup vs baseline: 1.0039x; 1.0039x over previous
import functools

import jax
import jax.numpy as jnp
from jax import lax
from jax.experimental import pallas as pl
from jax.experimental.pallas import tpu as pltpu
from jax.experimental.pallas import tpu_sc as plsc

_V = 1000
_D = 1000
_B, _T = 1024, 50
_NC, _NS = 2, 16
_NW = _NC * _NS
_BPW = _B // _NW


def _gather_body(table_hbm, idx_hbm, out_hbm, idx_v, rows_v, gsems, wsems):
    wid = lax.axis_index("s") * _NC + lax.axis_index("c")
    base = wid * _BPW
    pltpu.sync_copy(idx_hbm.at[pl.ds(base, _BPW)], idx_v)

    def fire_gather(ci):
        pltpu.async_copy(
            table_hbm.at[idx_v.at[ci]], rows_v.at[ci % 2], gsems.at[ci % 2]
        )

    def wait_gather(ci):
        pltpu.make_async_copy(
            table_hbm.at[idx_v.at[ci]], rows_v.at[ci % 2], gsems.at[ci % 2]
        ).wait()

    def fire_write(ci):
        pltpu.async_copy(
            rows_v.at[ci % 2], out_hbm.at[base + ci], wsems.at[ci % 2]
        )

    def wait_write(ci):
        pltpu.make_async_copy(
            rows_v.at[ci % 2], out_hbm.at[base + ci], wsems.at[ci % 2]
        ).wait()

    fire_gather(0)
    fire_gather(1)
    for ci in range(_BPW):
        wait_gather(ci)
        fire_write(ci)
        if ci + 2 < _BPW:
            wait_write(ci)
            fire_gather(ci + 2)
    wait_write(_BPW - 2)
    wait_write(_BPW - 1)


_mesh = plsc.VectorSubcoreMesh(core_axis_name="c", subcore_axis_name="s")

_gather = functools.partial(
    pl.kernel,
    out_type=jax.ShapeDtypeStruct((_B, _T, _D), jnp.float32),
    mesh=_mesh,
    scratch_types=[
        pltpu.VMEM((_BPW, _T), jnp.int32),
        pltpu.VMEM((2, _T, _D), jnp.float32),
        pltpu.SemaphoreType.DMA((2,)),
        pltpu.SemaphoreType.DMA((2,)),
    ],
    compiler_params=pltpu.CompilerParams(use_tc_tiling_on_sc=False),
)(_gather_body)


@jax.jit
def kernel(x, embeddings):
    return _gather(embeddings, x)


# speedup vs baseline: 1.3626x; 1.3573x over previous
import functools

import jax
import jax.numpy as jnp
from jax import lax
from jax.experimental import pallas as pl
from jax.experimental.pallas import tpu as pltpu
from jax.experimental.pallas import tpu_sc as plsc

_V = 1000
_D = 1000
_DP = 1024
_B, _T = 1024, 50
_N = _B * _T
_NC, _NS = 2, 16
_NW = _NC * _NS
_PERW = _N // _NW
_CH = 40
_NCHUNK = _PERW // _CH
_TAIL = _D - 896


def _gather_body(table_hbm, idx_hbm, out_hbm, idx_v, rows_v, tail_v, gsems, wsems):
    wid = lax.axis_index("s") * _NC + lax.axis_index("c")
    base = wid * _PERW
    pltpu.sync_copy(idx_hbm.at[pl.ds(base, _PERW)], idx_v)

    def fire_gather(ci, buf):
        off = pl.multiple_of(ci * _CH, 8)
        pltpu.async_copy(
            table_hbm.at[idx_v.at[pl.ds(off, _CH)]],
            rows_v.at[buf],
            gsems.at[buf],
        )

    def wait_gather(ci, buf):
        off = pl.multiple_of(ci * _CH, 8)
        pltpu.make_async_copy(
            table_hbm.at[idx_v.at[pl.ds(off, _CH)]],
            rows_v.at[buf],
            gsems.at[buf],
        ).wait()

    def repack_tail(buf):
        offs = [0, 16, 32, 48, 64, 80, 88]

        @pl.loop(0, _CH)
        def _(t):
            for off in offs:
                tail_v[buf, t, pl.ds(off, 16)] = rows_v[
                    buf, t, pl.ds(896 + off, 16)
                ]

    def fire_writes(ci, buf):
        row0 = pl.multiple_of(base + ci * _CH, 8)
        pltpu.async_copy(
            rows_v.at[buf, :, pl.ds(0, 896)],
            out_hbm.at[pl.ds(row0, _CH), pl.ds(0, 896)],
            wsems.at[buf],
        )
        pltpu.async_copy(
            tail_v.at[buf],
            out_hbm.at[pl.ds(row0, _CH), pl.ds(896, _TAIL)],
            wsems.at[buf],
        )

    def wait_writes(ci, buf):
        row0 = pl.multiple_of(base + ci * _CH, 8)
        pltpu.make_async_copy(
            rows_v.at[buf, :, pl.ds(0, 896)],
            out_hbm.at[pl.ds(row0, _CH), pl.ds(0, 896)],
            wsems.at[buf],
        ).wait()
        pltpu.make_async_copy(
            tail_v.at[buf],
            out_hbm.at[pl.ds(row0, _CH), pl.ds(896, _TAIL)],
            wsems.at[buf],
        ).wait()

    fire_gather(0, 0)
    fire_gather(1, 1)

    @pl.loop(0, _NCHUNK - 2, step=2)
    def _(ci):
        for buf in range(2):
            wait_gather(ci + buf, buf)
            repack_tail(buf)
            fire_writes(ci + buf, buf)
        for buf in range(2):
            wait_writes(ci + buf, buf)
            fire_gather(ci + 2 + buf, buf)

    for buf in range(2):
        wait_gather(_NCHUNK - 2 + buf, buf)
        repack_tail(buf)
        fire_writes(_NCHUNK - 2 + buf, buf)
    for buf in range(2):
        wait_writes(_NCHUNK - 2 + buf, buf)


_mesh = plsc.VectorSubcoreMesh(core_axis_name="c", subcore_axis_name="s")

_gather = functools.partial(
    pl.kernel,
    out_type=jax.ShapeDtypeStruct((_N, _D), jnp.float32),
    mesh=_mesh,
    scratch_types=[
        pltpu.VMEM((_PERW,), jnp.int32),
        pltpu.VMEM((2, _CH, _DP), jnp.float32),
        pltpu.VMEM((2, _CH, _TAIL), jnp.float32),
        pltpu.SemaphoreType.DMA((2,)),
        pltpu.SemaphoreType.DMA((2,)),
    ],
    compiler_params=pltpu.CompilerParams(use_tc_tiling_on_sc=True),
)(_gather_body)


@jax.jit
def kernel(x, embeddings):
    table = jnp.pad(embeddings, ((0, 0), (0, _DP - _D)))
    idx = x.reshape(_N)
    return _gather(table, idx).reshape(_B, _T, _D)


# speedup vs baseline: 1.3830x; 1.0150x over previous
import functools

import jax
import jax.numpy as jnp
from jax import lax
from jax.experimental import pallas as pl
from jax.experimental.pallas import tpu as pltpu
from jax.experimental.pallas import tpu_sc as plsc

_V = 1000
_D = 1000
_DP = 1024
_B, _T = 1024, 50
_TP = 56
_NC, _NS = 2, 16
_NW = _NC * _NS
_BPW = _B // _NW
_TAIL = _D - 896
_TF = 48


def _gather_body(table_hbm, idx_hbm, out_hbm, idx_v, rows_v, tail_v, gsems, wsems):
    wid = lax.axis_index("s") * _NC + lax.axis_index("c")
    base = wid * _BPW
    pltpu.sync_copy(idx_hbm.at[pl.ds(base * _TP, _BPW * _TP)], idx_v)

    def fire_gather(ci, buf):
        off = pl.multiple_of(ci * _TP, 8)
        pltpu.async_copy(
            table_hbm.at[idx_v.at[pl.ds(off, _TF)]],
            rows_v.at[buf],
            gsems.at[buf],
        )

    def wait_gather(ci, buf):
        off = pl.multiple_of(ci * _TP, 8)
        pltpu.make_async_copy(
            table_hbm.at[idx_v.at[pl.ds(off, _TF)]],
            rows_v.at[buf],
            gsems.at[buf],
        ).wait()

    def repack_tail(buf):
        @pl.loop(0, _TF)
        def _(t):
            for off in (0, 16, 32, 48, 64, 80, 88):
                tail_v[buf, t, pl.ds(off, 16)] = rows_v[
                    buf, t, pl.ds(896 + off, 16)
                ]

    def fire_writes(ci, buf):
        pltpu.async_copy(
            rows_v.at[buf, :, pl.ds(0, 896)],
            out_hbm.at[base + ci, pl.ds(0, _TF), pl.ds(0, 896)],
            wsems.at[buf],
        )
        pltpu.async_copy(
            tail_v.at[buf],
            out_hbm.at[base + ci, pl.ds(0, _TF), pl.ds(896, _TAIL)],
            wsems.at[buf],
        )

    def wait_writes(ci, buf):
        pltpu.make_async_copy(
            rows_v.at[buf, :, pl.ds(0, 896)],
            out_hbm.at[base + ci, pl.ds(0, _TF), pl.ds(0, 896)],
            wsems.at[buf],
        ).wait()
        pltpu.make_async_copy(
            tail_v.at[buf],
            out_hbm.at[base + ci, pl.ds(0, _TF), pl.ds(896, _TAIL)],
            wsems.at[buf],
        ).wait()

    fire_gather(0, 0)
    fire_gather(1, 1)

    @pl.loop(0, _BPW - 2, step=2)
    def _(ci):
        for buf in range(2):
            wait_gather(ci + buf, buf)
            repack_tail(buf)
            fire_writes(ci + buf, buf)
        for buf in range(2):
            wait_writes(ci + buf, buf)
            fire_gather(ci + 2 + buf, buf)

    for buf in range(2):
        wait_gather(_BPW - 2 + buf, buf)
        repack_tail(buf)
        fire_writes(_BPW - 2 + buf, buf)
    for buf in range(2):
        wait_writes(_BPW - 2 + buf, buf)


_mesh = plsc.VectorSubcoreMesh(core_axis_name="c", subcore_axis_name="s")

_gather = functools.partial(
    pl.kernel,
    out_type=jax.ShapeDtypeStruct((_B, _T, _D), jnp.float32),
    mesh=_mesh,
    scratch_types=[
        pltpu.VMEM((_BPW * _TP,), jnp.int32),
        pltpu.VMEM((2, _TF, _DP), jnp.float32),
        pltpu.VMEM((2, _TF, _TAIL), jnp.float32),
        pltpu.SemaphoreType.DMA((2,)),
        pltpu.SemaphoreType.DMA((2,)),
    ],
    compiler_params=pltpu.CompilerParams(use_tc_tiling_on_sc=True),
)(_gather_body)


def _fix_body(o_in_ref, x_ref, emb_ref, o_ref, scratch, sem):
    del o_in_ref
    i = pl.program_id(0)
    for r in range(8):
        for c in range(2):
            scratch[r, c] = emb_ref[x_ref[r, c]]
    pltpu.async_copy(
        scratch, o_ref.at[pl.ds(i * 8, 8), pl.ds(_TF, _T - _TF)], sem
    ).wait()


_fixup = pl.pallas_call(
    _fix_body,
    grid=(_B // 8,),
    in_specs=[
        pl.BlockSpec(memory_space=pl.ANY),
        pl.BlockSpec((8, 2), lambda i: (i, 0), memory_space=pltpu.SMEM),
        pl.BlockSpec((_V, _D), lambda i: (0, 0)),
    ],
    out_specs=pl.BlockSpec(memory_space=pl.ANY),
    out_shape=jax.ShapeDtypeStruct((_B, _T, _D), jnp.float32),
    scratch_shapes=[
        pltpu.VMEM((8, 2, _D), jnp.float32),
        pltpu.SemaphoreType.DMA,
    ],
    input_output_aliases={0: 0},
)


@jax.jit
def kernel(x, embeddings):
    table = jnp.pad(embeddings, ((0, 0), (0, _DP - _D)))
    idx = jnp.pad(x, ((0, 0), (0, _TP - _T))).reshape(_B * _TP)
    bulk = _gather(table, idx)
    return _fixup(bulk, x[:, _TF:_T], embeddings)


# speedup vs baseline: 1.6152x; 1.1679x over previous
import functools

import jax
import jax.numpy as jnp
from jax import lax
from jax.experimental import pallas as pl
from jax.experimental.pallas import tpu as pltpu
from jax.experimental.pallas import tpu_sc as plsc

_V = 1000
_D = 1000
_DP = 1024
_B, _T = 1024, 50
_TP = 56
_NC, _NS = 2, 16
_NW = _NC * _NS
_BPW = _B // _NW
_TAIL = _D - 896
_TF = 48


def _gather_body(table_hbm, idx_hbm, out_hbm, idx_v, rows_v, tail_v, gsems, wsems):
    wid = lax.axis_index("s") * _NC + lax.axis_index("c")
    base = wid * _BPW
    pltpu.sync_copy(idx_hbm.at[pl.ds(base * _TP, _BPW * _TP)], idx_v)

    def fire_gather(ci, buf):
        off = pl.multiple_of(ci * _TP, 8)
        pltpu.async_copy(
            table_hbm.at[idx_v.at[pl.ds(off, _TF)]],
            rows_v.at[buf],
            gsems.at[buf],
        )

    def wait_gather(ci, buf):
        off = pl.multiple_of(ci * _TP, 8)
        pltpu.make_async_copy(
            table_hbm.at[idx_v.at[pl.ds(off, _TF)]],
            rows_v.at[buf],
            gsems.at[buf],
        ).wait()

    def repack_tail(buf):
        @pl.loop(0, _TF)
        def _(t):
            for off in (0, 16, 32, 48, 64, 80, 88):
                tail_v[buf, t, pl.ds(off, 16)] = rows_v[
                    buf, t, pl.ds(896 + off, 16)
                ]

    def fire_writes(ci, buf):
        pltpu.async_copy(
            rows_v.at[buf, :, pl.ds(0, 896)],
            out_hbm.at[base + ci, pl.ds(0, _TF), pl.ds(0, 896)],
            wsems.at[buf],
        )
        pltpu.async_copy(
            tail_v.at[buf],
            out_hbm.at[base + ci, pl.ds(0, _TF), pl.ds(896, _TAIL)],
            wsems.at[buf],
        )

    def wait_writes(ci, buf):
        pltpu.make_async_copy(
            rows_v.at[buf, :, pl.ds(0, 896)],
            out_hbm.at[base + ci, pl.ds(0, _TF), pl.ds(0, 896)],
            wsems.at[buf],
        ).wait()
        pltpu.make_async_copy(
            tail_v.at[buf],
            out_hbm.at[base + ci, pl.ds(0, _TF), pl.ds(896, _TAIL)],
            wsems.at[buf],
        ).wait()

    fire_gather(0, 0)
    fire_gather(1, 1)

    @pl.loop(0, _BPW - 2, step=2)
    def _(ci):
        for buf in range(2):
            wait_gather(ci + buf, buf)
            repack_tail(buf)
            fire_writes(ci + buf, buf)
        for buf in range(2):
            wait_writes(ci + buf, buf)
            fire_gather(ci + 2 + buf, buf)

    for buf in range(2):
        wait_gather(_BPW - 2 + buf, buf)
        repack_tail(buf)
        fire_writes(_BPW - 2 + buf, buf)
    for buf in range(2):
        wait_writes(_BPW - 2 + buf, buf)


_mesh = plsc.VectorSubcoreMesh(core_axis_name="c", subcore_axis_name="s")

_gather = functools.partial(
    pl.kernel,
    out_type=jax.ShapeDtypeStruct((_B, _T, _D), jnp.float32),
    mesh=_mesh,
    scratch_types=[
        pltpu.VMEM((_BPW * _TP,), jnp.int32),
        pltpu.VMEM((2, _TF, _DP), jnp.float32),
        pltpu.VMEM((2, _TF, _TAIL), jnp.float32),
        pltpu.SemaphoreType.DMA((2,)),
        pltpu.SemaphoreType.DMA((2,)),
    ],
    compiler_params=pltpu.CompilerParams(use_tc_tiling_on_sc=True),
)(_gather_body)


_FIXB = 64


def _fix_body(o_in_ref, rows_ref, o_ref, sem):
    del o_in_ref
    i = pl.program_id(0)
    pltpu.async_copy(
        rows_ref, o_ref.at[pl.ds(i * _FIXB, _FIXB), pl.ds(_TF, _T - _TF)], sem
    ).wait()


_fixup = pl.pallas_call(
    _fix_body,
    grid=(_B // _FIXB,),
    in_specs=[
        pl.BlockSpec(memory_space=pl.ANY),
        pl.BlockSpec((_FIXB, _T - _TF, _D), lambda i: (i, 0, 0)),
    ],
    out_specs=pl.BlockSpec(memory_space=pl.ANY),
    out_shape=jax.ShapeDtypeStruct((_B, _T, _D), jnp.float32),
    scratch_shapes=[
        pltpu.SemaphoreType.DMA,
    ],
    input_output_aliases={0: 0},
)


@jax.jit
def kernel(x, embeddings):
    table = jnp.pad(embeddings, ((0, 0), (0, _DP - _D)))
    idx = jnp.pad(x, ((0, 0), (0, _TP - _T))).reshape(_B * _TP)
    bulk = _gather(table, idx)
    fix_rows = jnp.take(embeddings, x[:, _TF:_T], axis=0)
    return _fixup(bulk, fix_rows)
